# baseline (speedup 1.0000x reference)
import jax
import jax.numpy as jnp
from jax import lax
from jax.experimental import pallas as pl
from jax.experimental.pallas import tpu as pltpu
from jax.experimental.pallas import tpu_sc as plsc

B = 16384
D = 32
NR = 1000
NW = 32
BPW = B // NW
CH = 16
NCH = BPW // CH


def _trig_body(p_ref, cos_ref, sin_ref):
    p = p_ref[...]
    cos_ref[...] = jnp.cos(p)
    sin_ref[...] = jnp.sin(p)


def _trig_tables(relation_phase):
    out = jax.ShapeDtypeStruct((NR, D), jnp.float32)
    return pl.pallas_call(
        _trig_body,
        out_shape=(out, out),
    )(relation_phase)


def _score_body(dsq_ref, out_ref):
    out_ref[...] = -jnp.sum(jnp.sqrt(dsq_ref[...]), axis=-1)


def _score(dsq):
    return pl.pallas_call(
        _score_body,
        out_shape=jax.ShapeDtypeStruct((B,), jnp.float32),
    )(dsq)


def _sc_body(h_hbm, r_hbm, t_hbm, ere_hbm, eim_hbm, cos_hbm, sin_hbm,
             out_hbm, hi_v, ti_v, r_v, bufs_a, bufs_b, dsq_v,
             sem_a, sem_b):
    c = lax.axis_index("c")
    s = lax.axis_index("s")
    wid = s * 2 + c
    base = wid * BPW

    pltpu.sync_copy(h_hbm.at[pl.ds(base, BPW)], hi_v)
    pltpu.sync_copy(t_hbm.at[pl.ds(base, BPW)], ti_v)
    pltpu.sync_copy(r_hbm.at[pl.ds(base, BPW)], r_v)

    def fire(cidx, bufs, sem):
        hvec = hi_v[pl.ds(cidx * CH, CH)]
        tvec = ti_v[pl.ds(cidx * CH, CH)]
        rvec = r_v[pl.ds(cidx * CH, CH)]
        for k in range(CH):
            pltpu.async_copy(ere_hbm.at[hvec[k]], bufs[0].at[k], sem)
            pltpu.async_copy(eim_hbm.at[hvec[k]], bufs[1].at[k], sem)
            pltpu.async_copy(ere_hbm.at[tvec[k]], bufs[2].at[k], sem)
            pltpu.async_copy(eim_hbm.at[tvec[k]], bufs[3].at[k], sem)
            pltpu.async_copy(cos_hbm.at[rvec[k]], bufs[4].at[k], sem)
            pltpu.async_copy(sin_hbm.at[rvec[k]], bufs[5].at[k], sem)

    def drain(bufs, sem):
        @pl.loop(0, CH)
        def _(k):
            for b in range(6):
                pltpu.make_async_copy(ere_hbm.at[0], bufs[b].at[k], sem).wait()

    def compute(cidx, bufs):
        for k in range(CH):
            row = cidx * CH + k
            for cc in range(2):
                o = cc * 16
                hre = bufs[0][k, pl.ds(o, 16)]
                him = bufs[1][k, pl.ds(o, 16)]
                tre = bufs[2][k, pl.ds(o, 16)]
                tim = bufs[3][k, pl.ds(o, 16)]
                rre = bufs[4][k, pl.ds(o, 16)]
                rim = bufs[5][k, pl.ds(o, 16)]
                dre = hre * rre - him * rim - tre
                dim = hre * rim + him * rre - tim
                dsq_v[row, pl.ds(o, 16)] = dre * dre + dim * dim + 1e-12

    fire(0, bufs_a, sem_a)

    @pl.loop(0, NCH, step=2)
    def _(c0):
        fire(c0 + 1, bufs_b, sem_b)
        drain(bufs_a, sem_a)
        compute(c0, bufs_a)

        @pl.when(c0 + 2 < NCH)
        def _():
            fire(c0 + 2, bufs_a, sem_a)

        drain(bufs_b, sem_b)
        compute(c0 + 1, bufs_b)

    pltpu.sync_copy(dsq_v, out_hbm.at[pl.ds(base, BPW)])


def _sc_dsq(heads, relations, tails, entity_re, entity_im, cos_t, sin_t):
    mesh = plsc.VectorSubcoreMesh(core_axis_name="c", subcore_axis_name="s")
    buf = pltpu.VMEM((CH, D), jnp.float32)
    fn = pl.kernel(
        _sc_body,
        out_type=jax.ShapeDtypeStruct((B, D), jnp.float32),
        mesh=mesh,
        compiler_params=pltpu.CompilerParams(use_tc_tiling_on_sc=False),
        scratch_types=[
            pltpu.VMEM((BPW,), jnp.int32),
            pltpu.VMEM((BPW,), jnp.int32),
            pltpu.VMEM((BPW,), jnp.int32),
            [buf] * 6,
            [buf] * 6,
            pltpu.VMEM((BPW, D), jnp.float32),
            pltpu.SemaphoreType.DMA,
            pltpu.SemaphoreType.DMA,
        ],
    )
    return fn(heads, relations, tails, entity_re, entity_im, cos_t, sin_t)


def kernel(heads, relations, tails, entity_re, entity_im, relation_phase):
    cos_t, sin_t = _trig_tables(relation_phase)
    dsq = _sc_dsq(heads.astype(jnp.int32), relations.astype(jnp.int32),
                  tails.astype(jnp.int32), entity_re, entity_im, cos_t, sin_t)
    return _score(dsq)


# speedup vs baseline: 1.5117x; 1.5117x over previous
import jax
import jax.numpy as jnp
from jax import lax
from jax.experimental import pallas as pl
from jax.experimental.pallas import tpu as pltpu
from jax.experimental.pallas import tpu_sc as plsc

B = 16384
D = 32
NR = 1000
NW = 32
BPW = B // NW
CH = 16
NCH = BPW // CH


def _trig_body(p_ref, cos_ref, sin_ref):
    p = p_ref[...]
    cos_ref[...] = jnp.cos(p)
    sin_ref[...] = jnp.sin(p)


def _trig_tables(relation_phase):
    out = jax.ShapeDtypeStruct((NR, D), jnp.float32)
    return pl.pallas_call(
        _trig_body,
        out_shape=(out, out),
    )(relation_phase)


def _sqrt_nr(x):
    u = plsc.bitcast(x, jnp.int32)
    y = plsc.bitcast(0x5F3759DF - (u >> 1), jnp.float32)
    xh = 0.5 * x
    y = y * (1.5 - xh * y * y)
    y = y * (1.5 - xh * y * y)
    y = y * (1.5 - xh * y * y)
    return x * y


def _sc_body(h_hbm, r_hbm, t_hbm, ere_hbm, eim_hbm, cos_hbm, sin_hbm,
             out_hbm, hi_v, ti_v, r_v, bufs_a, bufs_b, red_v, score_v,
             sem_a, sem_b):
    c = lax.axis_index("c")
    s = lax.axis_index("s")
    wid = s * 2 + c
    base = wid * BPW

    pltpu.sync_copy(h_hbm.at[pl.ds(base, BPW)], hi_v)
    pltpu.sync_copy(t_hbm.at[pl.ds(base, BPW)], ti_v)
    pltpu.sync_copy(r_hbm.at[pl.ds(base, BPW)], r_v)

    def fire(cidx, bufs, sem):
        hvec = hi_v[pl.ds(cidx * CH, CH)]
        tvec = ti_v[pl.ds(cidx * CH, CH)]
        rvec = r_v[pl.ds(cidx * CH, CH)]
        for k in range(CH):
            pltpu.async_copy(ere_hbm.at[hvec[k]], bufs[0].at[k], sem)
            pltpu.async_copy(eim_hbm.at[hvec[k]], bufs[1].at[k], sem)
            pltpu.async_copy(ere_hbm.at[tvec[k]], bufs[2].at[k], sem)
            pltpu.async_copy(eim_hbm.at[tvec[k]], bufs[3].at[k], sem)
            pltpu.async_copy(cos_hbm.at[rvec[k]], bufs[4].at[k], sem)
            pltpu.async_copy(sin_hbm.at[rvec[k]], bufs[5].at[k], sem)

    def drain(bufs, sem):
        @pl.loop(0, CH)
        def _(k):
            for b in range(6):
                pltpu.make_async_copy(ere_hbm.at[0], bufs[b].at[k], sem).wait()

    lanes = lax.iota(jnp.int32, 16)

    def compute(cidx, bufs):
        svec = jnp.zeros((16,), jnp.float32)
        for k in range(CH):
            acc = jnp.zeros((16,), jnp.float32)
            for cc in range(2):
                o = cc * 16
                hre = bufs[0][k, pl.ds(o, 16)]
                him = bufs[1][k, pl.ds(o, 16)]
                tre = bufs[2][k, pl.ds(o, 16)]
                tim = bufs[3][k, pl.ds(o, 16)]
                rre = bufs[4][k, pl.ds(o, 16)]
                rim = bufs[5][k, pl.ds(o, 16)]
                dre = hre * rre - him * rim - tre
                dim = hre * rim + him * rre - tim
                acc = acc + _sqrt_nr(dre * dre + dim * dim + 1e-12)
            for sh in (8, 4, 2, 1):
                red_v[k, pl.ds(0, 16)] = acc
                acc = acc + red_v[k, pl.ds(sh, 16)]
            svec = jnp.where(lanes == k, acc[0], svec)
        score_v[pl.ds(cidx * CH, CH)] = -svec

    fire(0, bufs_a, sem_a)

    @pl.loop(0, NCH, step=2)
    def _(c0):
        fire(c0 + 1, bufs_b, sem_b)
        drain(bufs_a, sem_a)
        compute(c0, bufs_a)

        @pl.when(c0 + 2 < NCH)
        def _():
            fire(c0 + 2, bufs_a, sem_a)

        drain(bufs_b, sem_b)
        compute(c0 + 1, bufs_b)

    pltpu.sync_copy(score_v, out_hbm.at[pl.ds(base, BPW)])


def _sc_dsq(heads, relations, tails, entity_re, entity_im, cos_t, sin_t):
    mesh = plsc.VectorSubcoreMesh(core_axis_name="c", subcore_axis_name="s")
    buf = pltpu.VMEM((CH, D), jnp.float32)
    fn = pl.kernel(
        _sc_body,
        out_type=jax.ShapeDtypeStruct((B,), jnp.float32),
        mesh=mesh,
        compiler_params=pltpu.CompilerParams(needs_layout_passes=False),
        scratch_types=[
            pltpu.VMEM((BPW,), jnp.int32),
            pltpu.VMEM((BPW,), jnp.int32),
            pltpu.VMEM((BPW,), jnp.int32),
            [buf] * 6,
            [buf] * 6,
            pltpu.VMEM((CH, 32), jnp.float32),
            pltpu.VMEM((BPW,), jnp.float32),
            pltpu.SemaphoreType.DMA,
            pltpu.SemaphoreType.DMA,
        ],
    )
    return fn(heads, relations, tails, entity_re, entity_im, cos_t, sin_t)


def kernel(heads, relations, tails, entity_re, entity_im, relation_phase):
    cos_t, sin_t = _trig_tables(relation_phase)
    return _sc_dsq(heads.astype(jnp.int32), relations.astype(jnp.int32),
                   tails.astype(jnp.int32), entity_re, entity_im, cos_t, sin_t)
